# baseline (speedup 1.0000x reference)
import jax
import jax.numpy as jnp
from jax import lax
from jax.experimental import pallas as pl
from jax.experimental.pallas import tpu as pltpu
from jax.experimental.pallas import tpu_sc as plsc

B = 4
N = 150
C = 150
P = N * (N - 1)
W = 2 * (4 + C + C + 3)

NP = 160
WV = 624
NSLOT = 8
CUT = 21456
RPW = 2688
RPT = CUT - 7 * RPW
IPB = 8
RPB = IPB * (N - 1)
TIDX = CUT // RPB

_XR = ((0, 4), (8, 158), (308, 458), (608, 609), (610, 611), (612, 613))
_YR = ((4, 8), (158, 308), (458, 608), (609, 610), (611, 612), (613, 614))


def _kind(off):
    if any(a <= off and off + 16 <= b for a, b in _XR):
        return 'x'
    if any(a <= off and off + 16 <= b for a, b in _YR):
        return 'y'
    return 'm'


CHUNKS = tuple((off, _kind(off))
               for off in tuple(range(0, 608, 16)) + (598,))


def _feature_rows(logits, box):
    s = jax.nn.sigmoid(logits)
    soft = jnp.minimum(1.0 - s, s)
    m = jnp.max(s, axis=-1, keepdims=True)
    lp = jnp.log(m + 1e-08)
    ln = jnp.log(1.0 - m + 1e-08)
    z4 = jnp.zeros((N, 4), jnp.float32)
    zC = jnp.zeros((N, C), jnp.float32)
    z1 = jnp.zeros((N, 1), jnp.float32)
    rx = jnp.concatenate(
        [box, z4, s, zC, soft, zC, lp, z1, ln, z1, m, z1], axis=-1)
    ry = jnp.concatenate(
        [z4, box, zC, s, zC, soft, z1, lp, z1, ln, z1, m], axis=-1)
    return rx, ry


def _table_body(logits_ref, boxes_ref, rx_ref, ry_ref):
    rx, ry = _feature_rows(logits_ref[0], boxes_ref[0])
    zpad = jnp.zeros((N, WV - W), jnp.float32)
    rx_ref[0, 0:N, :] = jnp.concatenate([rx, zpad], axis=-1)
    ry_ref[0, 0:N, :] = jnp.concatenate([ry, zpad], axis=-1)
    rx_ref[0, N:NP, :] = jnp.zeros((NP - N, WV), jnp.float32)
    ry_ref[0, N:NP, :] = jnp.zeros((NP - N, WV), jnp.float32)


def _sc_expand(rxp_ref, ryp_ref, out_ref, ry_v, rx_v, stage_v, sems):
    ci = lax.axis_index("c")
    si = lax.axis_index("s")
    wid = ci * 16 + si
    b = wid // NSLOT
    slot = wid % NSLOT
    r0 = slot * RPW
    nch = jnp.where(slot == NSLOT - 1, RPT // 8, RPW // 8)
    ix0 = pl.multiple_of((r0 // (N - 1)) // 8 * 8, 8)

    pltpu.sync_copy(ryp_ref.at[b, pl.ds(0, 152), :], ry_v)
    pltpu.sync_copy(rxp_ref.at[b, pl.ds(ix0, 32), :], rx_v)

    def chunk_fn(m, carry):
        mq = lax.rem(m, 2)
        rbase = pl.multiple_of(r0 + m * 8, 8)

        @pl.when(m >= 2)
        def _wait_reuse():
            pltpu.make_async_copy(
                stage_v.at[mq],
                out_ref.at[b, pl.ds(rbase, 8), :],
                sems.at[mq],
            ).wait()

        def row_fn(q, c2):
            r = rbase + q
            i = r // (N - 1)
            j = r - i * (N - 1)
            sry = j + (j >= i).astype(jnp.int32)
            srx = i - ix0
            for off, kind in CHUNKS:
                if kind == 'x':
                    v = rx_v[srx, pl.ds(off, 16)]
                elif kind == 'y':
                    v = ry_v[sry, pl.ds(off, 16)]
                else:
                    v = (rx_v[srx, pl.ds(off, 16)]
                         + ry_v[sry, pl.ds(off, 16)])
                stage_v[mq, q, pl.ds(off, 16)] = v
            return c2

        lax.fori_loop(0, 8, row_fn, 0)
        pltpu.make_async_copy(
            stage_v.at[mq],
            out_ref.at[b, pl.ds(rbase, 8), :],
            sems.at[mq],
        ).start()
        return carry

    lax.fori_loop(0, nch, chunk_fn, 0)
    for mq in range(2):
        pltpu.make_async_copy(
            stage_v.at[mq],
            out_ref.at[b, pl.ds(r0, 8), :],
            sems.at[mq],
        ).wait()


def _tail_body(logits_ref, boxes_ref, _big_ref, out_ref):
    rx, ry = _feature_rows(logits_ref[0], boxes_ref[0])
    lo = ry[0:N - 1, :]
    hi = ry[1:N, :]
    k = lax.broadcasted_iota(jnp.int32, (N - 1, W), 0)
    for di in range(IPB):
        i = min(TIDX * IPB + di, N - 1)
        rowx = rx[i, :].reshape(1, W)
        sub = jnp.where(k < i, lo, hi) + rowx
        out_ref[0, pl.ds(di * (N - 1), N - 1), :] = sub


def kernel(class_logits, pred_bboxes):
    rxp, ryp = pl.pallas_call(
        _table_body,
        grid=(B,),
        in_specs=[
            pl.BlockSpec((1, N, C), lambda b: (b, 0, 0)),
            pl.BlockSpec((1, N, 4), lambda b: (b, 0, 0)),
        ],
        out_specs=[
            pl.BlockSpec((1, NP, WV), lambda b: (b, 0, 0)),
            pl.BlockSpec((1, NP, WV), lambda b: (b, 0, 0)),
        ],
        out_shape=[
            jax.ShapeDtypeStruct((B, NP, WV), jnp.float32),
            jax.ShapeDtypeStruct((B, NP, WV), jnp.float32),
        ],
    )(class_logits, pred_bboxes)

    sc_expand = pl.kernel(
        _sc_expand,
        out_type=jax.ShapeDtypeStruct((B, P, W), jnp.float32),
        mesh=plsc.VectorSubcoreMesh(core_axis_name="c", subcore_axis_name="s"),
        scratch_types=[
            pltpu.VMEM((152, WV), jnp.float32),
            pltpu.VMEM((32, WV), jnp.float32),
            pltpu.VMEM((2, 8, W), jnp.float32),
            pltpu.SemaphoreType.DMA((2,)),
        ],
    )
    big = sc_expand(rxp, ryp)

    out = pl.pallas_call(
        _tail_body,
        grid=(B,),
        in_specs=[
            pl.BlockSpec((1, N, C), lambda b: (b, 0, 0)),
            pl.BlockSpec((1, N, 4), lambda b: (b, 0, 0)),
            pl.BlockSpec(memory_space=pl.ANY),
        ],
        out_specs=pl.BlockSpec((1, RPB, W), lambda b: (b, TIDX, 0)),
        out_shape=jax.ShapeDtypeStruct((B, P, W), jnp.float32),
        input_output_aliases={2: 0},
    )(class_logits, pred_bboxes, big)
    return out


# speedup vs baseline: 1.0011x; 1.0011x over previous
import jax
import jax.numpy as jnp
from jax import lax
from jax.experimental import pallas as pl
from jax.experimental.pallas import tpu as pltpu
from jax.experimental.pallas import tpu_sc as plsc

B = 4
N = 150
C = 150
P = N * (N - 1)
W = 2 * (4 + C + C + 3)

NP = 160
WV = 624
NSLOT = 8
CUT = 21456
RPW = 2688
RPT = CUT - 7 * RPW
IPB = 8
RPB = IPB * (N - 1)
TIDX = CUT // RPB

_XR = ((0, 4), (8, 158), (308, 458), (608, 609), (610, 611), (612, 613))
_YR = ((4, 8), (158, 308), (458, 608), (609, 610), (611, 612), (613, 614))


def _kind(off):
    if any(a <= off and off + 16 <= b for a, b in _XR):
        return 'x'
    if any(a <= off and off + 16 <= b for a, b in _YR):
        return 'y'
    return 'm'


CHUNKS = tuple((off, _kind(off))
               for off in tuple(range(0, 608, 16)) + (598,))


def _feature_rows(logits, box):
    s = jax.nn.sigmoid(logits)
    soft = jnp.minimum(1.0 - s, s)
    m = jnp.max(s, axis=-1, keepdims=True)
    lp = jnp.log(m + 1e-08)
    ln = jnp.log(1.0 - m + 1e-08)
    z4 = jnp.zeros((N, 4), jnp.float32)
    zC = jnp.zeros((N, C), jnp.float32)
    z1 = jnp.zeros((N, 1), jnp.float32)
    rx = jnp.concatenate(
        [box, z4, s, zC, soft, zC, lp, z1, ln, z1, m, z1], axis=-1)
    ry = jnp.concatenate(
        [z4, box, zC, s, zC, soft, z1, lp, z1, ln, z1, m], axis=-1)
    return rx, ry


def _table_body(logits_ref, boxes_ref, rx_ref, ry_ref):
    rx, ry = _feature_rows(logits_ref[0], boxes_ref[0])
    zpad = jnp.zeros((N, WV - W), jnp.float32)
    rx_ref[0, 0:N, :] = jnp.concatenate([rx, zpad], axis=-1)
    ry_ref[0, 0:N, :] = jnp.concatenate([ry, zpad], axis=-1)
    rx_ref[0, N:NP, :] = jnp.zeros((NP - N, WV), jnp.float32)
    ry_ref[0, N:NP, :] = jnp.zeros((NP - N, WV), jnp.float32)


def _sc_expand(rxp_ref, ryp_ref, out_ref, ry_v, rx_v, stage_v, sems):
    ci = lax.axis_index("c")
    si = lax.axis_index("s")
    wid = ci * 16 + si
    b = wid // NSLOT
    slot = wid % NSLOT
    r0 = slot * RPW
    nch = jnp.where(slot == NSLOT - 1, RPT // 8, RPW // 8)
    ix0 = pl.multiple_of((r0 // (N - 1)) // 8 * 8, 8)

    pltpu.sync_copy(ryp_ref.at[b, pl.ds(0, 152), :], ry_v)
    pltpu.sync_copy(rxp_ref.at[b, pl.ds(ix0, 32), :], rx_v)

    def chunk_fn(m, carry):
        mq = lax.rem(m, 2)
        rbase = pl.multiple_of(r0 + m * 8, 8)

        @pl.when(m >= 2)
        def _wait_reuse():
            pltpu.make_async_copy(
                stage_v.at[mq],
                out_ref.at[b, pl.ds(rbase, 8), :],
                sems.at[mq],
            ).wait()

        i0 = rbase // (N - 1)
        j0 = rbase - i0 * (N - 1)
        for q in range(8):
            w = (j0 + q >= N - 1).astype(jnp.int32)
            i = i0 + w
            j = j0 + q - w * (N - 1)
            sry = j + (j >= i).astype(jnp.int32)
            srx = i - ix0
            for off, kind in CHUNKS:
                if kind == 'x':
                    v = rx_v[srx, pl.ds(off, 16)]
                elif kind == 'y':
                    v = ry_v[sry, pl.ds(off, 16)]
                else:
                    v = (rx_v[srx, pl.ds(off, 16)]
                         + ry_v[sry, pl.ds(off, 16)])
                stage_v[mq, q, pl.ds(off, 16)] = v
        pltpu.make_async_copy(
            stage_v.at[mq],
            out_ref.at[b, pl.ds(rbase, 8), :],
            sems.at[mq],
        ).start()
        return carry

    lax.fori_loop(0, nch, chunk_fn, 0)
    for mq in range(2):
        pltpu.make_async_copy(
            stage_v.at[mq],
            out_ref.at[b, pl.ds(r0, 8), :],
            sems.at[mq],
        ).wait()


def _tail_body(logits_ref, boxes_ref, _big_ref, out_ref):
    rx, ry = _feature_rows(logits_ref[0], boxes_ref[0])
    lo = ry[0:N - 1, :]
    hi = ry[1:N, :]
    k = lax.broadcasted_iota(jnp.int32, (N - 1, W), 0)
    for di in range(IPB):
        i = min(TIDX * IPB + di, N - 1)
        rowx = rx[i, :].reshape(1, W)
        sub = jnp.where(k < i, lo, hi) + rowx
        out_ref[0, pl.ds(di * (N - 1), N - 1), :] = sub


def kernel(class_logits, pred_bboxes):
    rxp, ryp = pl.pallas_call(
        _table_body,
        grid=(B,),
        in_specs=[
            pl.BlockSpec((1, N, C), lambda b: (b, 0, 0)),
            pl.BlockSpec((1, N, 4), lambda b: (b, 0, 0)),
        ],
        out_specs=[
            pl.BlockSpec((1, NP, WV), lambda b: (b, 0, 0)),
            pl.BlockSpec((1, NP, WV), lambda b: (b, 0, 0)),
        ],
        out_shape=[
            jax.ShapeDtypeStruct((B, NP, WV), jnp.float32),
            jax.ShapeDtypeStruct((B, NP, WV), jnp.float32),
        ],
    )(class_logits, pred_bboxes)

    sc_expand = pl.kernel(
        _sc_expand,
        out_type=jax.ShapeDtypeStruct((B, P, W), jnp.float32),
        mesh=plsc.VectorSubcoreMesh(core_axis_name="c", subcore_axis_name="s"),
        scratch_types=[
            pltpu.VMEM((152, WV), jnp.float32),
            pltpu.VMEM((32, WV), jnp.float32),
            pltpu.VMEM((2, 8, W), jnp.float32),
            pltpu.SemaphoreType.DMA((2,)),
        ],
    )
    big = sc_expand(rxp, ryp)

    out = pl.pallas_call(
        _tail_body,
        grid=(B,),
        in_specs=[
            pl.BlockSpec((1, N, C), lambda b: (b, 0, 0)),
            pl.BlockSpec((1, N, 4), lambda b: (b, 0, 0)),
            pl.BlockSpec(memory_space=pl.ANY),
        ],
        out_specs=pl.BlockSpec((1, RPB, W), lambda b: (b, TIDX, 0)),
        out_shape=jax.ShapeDtypeStruct((B, P, W), jnp.float32),
        input_output_aliases={2: 0},
    )(class_logits, pred_bboxes, big)
    return out


# speedup vs baseline: 1.6106x; 1.6087x over previous
import jax
import jax.numpy as jnp
from jax import lax
from jax.experimental import pallas as pl
from jax.experimental.pallas import tpu as pltpu
from jax.experimental.pallas import tpu_sc as plsc

B = 4
N = 150
C = 150
P = N * (N - 1)
W = 2 * (4 + C + C + 3)

NP = 160
WV = 624
NSLOT = 8
CUT = 21456
RPW = 2688
RPT = CUT - 7 * RPW
IPB = 8
RPB = IPB * (N - 1)
TIDX = CUT // RPB

_XR = ((0, 4), (8, 158), (308, 458), (608, 609), (610, 611), (612, 613))
_YR = ((4, 8), (158, 308), (458, 608), (609, 610), (611, 612), (613, 614))


def _kind(off):
    if any(a <= off and off + 16 <= b for a, b in _XR):
        return 'x'
    if any(a <= off and off + 16 <= b for a, b in _YR):
        return 'y'
    return 'm'


CHUNKS = tuple((off, _kind(off))
               for off in tuple(range(0, 608, 16)) + (598,))


def _feature_rows(logits, box):
    s = jax.nn.sigmoid(logits)
    soft = jnp.minimum(1.0 - s, s)
    m = jnp.max(s, axis=-1, keepdims=True)
    lp = jnp.log(m + 1e-08)
    ln = jnp.log(1.0 - m + 1e-08)
    z4 = jnp.zeros((N, 4), jnp.float32)
    zC = jnp.zeros((N, C), jnp.float32)
    z1 = jnp.zeros((N, 1), jnp.float32)
    rx = jnp.concatenate(
        [box, z4, s, zC, soft, zC, lp, z1, ln, z1, m, z1], axis=-1)
    ry = jnp.concatenate(
        [z4, box, zC, s, zC, soft, z1, lp, z1, ln, z1, m], axis=-1)
    return rx, ry


def _table_body(logits_ref, boxes_ref, rx_ref, ry_ref):
    rx, ry = _feature_rows(logits_ref[0], boxes_ref[0])
    zpad = jnp.zeros((N, WV - W), jnp.float32)
    rx_ref[0, 0:N, :] = jnp.concatenate([rx, zpad], axis=-1)
    ry_ref[0, 0:N, :] = jnp.concatenate([ry, zpad], axis=-1)
    rx_ref[0, N:NP, :] = jnp.zeros((NP - N, WV), jnp.float32)
    ry_ref[0, N:NP, :] = jnp.zeros((NP - N, WV), jnp.float32)


def _sc_expand(rxp_ref, ryp_ref, out_ref, ry_v, rx_v, stage_v, sems):
    ci = lax.axis_index("c")
    si = lax.axis_index("s")
    wid = ci * 16 + si
    b = wid // NSLOT
    slot = wid % NSLOT
    r0 = slot * RPW
    nch = jnp.where(slot == NSLOT - 1, RPT // 8, RPW // 8)
    ix0 = pl.multiple_of((r0 // (N - 1)) // 8 * 8, 8)

    pltpu.sync_copy(ryp_ref.at[b, pl.ds(0, 152), :], ry_v)
    pltpu.sync_copy(rxp_ref.at[b, pl.ds(ix0, 32), :], rx_v)

    def chunk_fn(m, carry):
        mq = lax.rem(m, 2)
        rbase = pl.multiple_of(r0 + m * 8, 8)

        @pl.when(m >= 2)
        def _wait_reuse():
            pltpu.make_async_copy(
                stage_v.at[mq],
                out_ref.at[b, pl.ds(rbase, 8), :],
                sems.at[mq],
            ).wait()

        i0 = rbase // (N - 1)
        j0 = rbase - i0 * (N - 1)
        for q in range(8):
            w = (j0 + q >= N - 1).astype(jnp.int32)
            i = i0 + w
            j = j0 + q - w * (N - 1)
            sry = j + (j >= i).astype(jnp.int32)
            srx = i - ix0
            vals = []
            for off, kind in CHUNKS:
                if kind == 'x':
                    v = rx_v[srx, pl.ds(off, 16)]
                elif kind == 'y':
                    v = ry_v[sry, pl.ds(off, 16)]
                else:
                    v = (rx_v[srx, pl.ds(off, 16)]
                         + ry_v[sry, pl.ds(off, 16)])
                vals.append(v)
            for (off, _), v in zip(CHUNKS, vals):
                stage_v[mq, q, pl.ds(off, 16)] = v
        pltpu.make_async_copy(
            stage_v.at[mq],
            out_ref.at[b, pl.ds(rbase, 8), :],
            sems.at[mq],
        ).start()
        return carry

    lax.fori_loop(0, nch, chunk_fn, 0)
    for mq in range(2):
        pltpu.make_async_copy(
            stage_v.at[mq],
            out_ref.at[b, pl.ds(r0, 8), :],
            sems.at[mq],
        ).wait()


def _tail_body(logits_ref, boxes_ref, _big_ref, out_ref):
    rx, ry = _feature_rows(logits_ref[0], boxes_ref[0])
    lo = ry[0:N - 1, :]
    hi = ry[1:N, :]
    k = lax.broadcasted_iota(jnp.int32, (N - 1, W), 0)
    for di in range(IPB):
        i = min(TIDX * IPB + di, N - 1)
        rowx = rx[i, :].reshape(1, W)
        sub = jnp.where(k < i, lo, hi) + rowx
        out_ref[0, pl.ds(di * (N - 1), N - 1), :] = sub


def kernel(class_logits, pred_bboxes):
    rxp, ryp = pl.pallas_call(
        _table_body,
        grid=(B,),
        in_specs=[
            pl.BlockSpec((1, N, C), lambda b: (b, 0, 0)),
            pl.BlockSpec((1, N, 4), lambda b: (b, 0, 0)),
        ],
        out_specs=[
            pl.BlockSpec((1, NP, WV), lambda b: (b, 0, 0)),
            pl.BlockSpec((1, NP, WV), lambda b: (b, 0, 0)),
        ],
        out_shape=[
            jax.ShapeDtypeStruct((B, NP, WV), jnp.float32),
            jax.ShapeDtypeStruct((B, NP, WV), jnp.float32),
        ],
    )(class_logits, pred_bboxes)

    sc_expand = pl.kernel(
        _sc_expand,
        out_type=jax.ShapeDtypeStruct((B, P, W), jnp.float32),
        mesh=plsc.VectorSubcoreMesh(core_axis_name="c", subcore_axis_name="s"),
        scratch_types=[
            pltpu.VMEM((152, WV), jnp.float32),
            pltpu.VMEM((32, WV), jnp.float32),
            pltpu.VMEM((2, 8, W), jnp.float32),
            pltpu.SemaphoreType.DMA((2,)),
        ],
    )
    big = sc_expand(rxp, ryp)

    out = pl.pallas_call(
        _tail_body,
        grid=(B,),
        in_specs=[
            pl.BlockSpec((1, N, C), lambda b: (b, 0, 0)),
            pl.BlockSpec((1, N, 4), lambda b: (b, 0, 0)),
            pl.BlockSpec(memory_space=pl.ANY),
        ],
        out_specs=pl.BlockSpec((1, RPB, W), lambda b: (b, TIDX, 0)),
        out_shape=jax.ShapeDtypeStruct((B, P, W), jnp.float32),
        input_output_aliases={2: 0},
    )(class_logits, pred_bboxes, big)
    return out


# speedup vs baseline: 1.8204x; 1.1303x over previous
import jax
import jax.numpy as jnp
from jax import lax
from jax.experimental import pallas as pl
from jax.experimental.pallas import tpu as pltpu
from jax.experimental.pallas import tpu_sc as plsc

B = 4
N = 150
C = 150
P = N * (N - 1)
W = 2 * (4 + C + C + 3)

NP = 160
WV = 624
NSLOT = 8
CUT = 21456
RPW = 2688
RPT = CUT - 7 * RPW
IPB = 8
RPB = IPB * (N - 1)
TIDX = CUT // RPB

_XR = ((0, 4), (8, 158), (308, 458), (608, 609), (610, 611), (612, 613))
_YR = ((4, 8), (158, 308), (458, 608), (609, 610), (611, 612), (613, 614))


def _kind(off):
    if any(a <= off and off + 16 <= b for a, b in _XR):
        return 'x'
    if any(a <= off and off + 16 <= b for a, b in _YR):
        return 'y'
    return 'm'


CHUNKS = tuple((off, _kind(off))
               for off in tuple(range(0, 608, 16)) + (598,))


def _feature_rows(logits, box):
    s = jax.nn.sigmoid(logits)
    soft = jnp.minimum(1.0 - s, s)
    m = jnp.max(s, axis=-1, keepdims=True)
    lp = jnp.log(m + 1e-08)
    ln = jnp.log(1.0 - m + 1e-08)
    z4 = jnp.zeros((N, 4), jnp.float32)
    zC = jnp.zeros((N, C), jnp.float32)
    z1 = jnp.zeros((N, 1), jnp.float32)
    rx = jnp.concatenate(
        [box, z4, s, zC, soft, zC, lp, z1, ln, z1, m, z1], axis=-1)
    ry = jnp.concatenate(
        [z4, box, zC, s, zC, soft, z1, lp, z1, ln, z1, m], axis=-1)
    return rx, ry


def _table_body(logits_ref, boxes_ref, rx_ref, ry_ref):
    rx, ry = _feature_rows(logits_ref[0], boxes_ref[0])
    zpad = jnp.zeros((N, WV - W), jnp.float32)
    rx_ref[0, 0:N, :] = jnp.concatenate([rx, zpad], axis=-1)
    ry_ref[0, 0:N, :] = jnp.concatenate([ry, zpad], axis=-1)
    rx_ref[0, N:NP, :] = jnp.zeros((NP - N, WV), jnp.float32)
    ry_ref[0, N:NP, :] = jnp.zeros((NP - N, WV), jnp.float32)


def _sc_expand(rxp_ref, ryp_ref, out_ref, ry_v, rx_v, stage_v, sems):
    ci = lax.axis_index("c")
    si = lax.axis_index("s")
    wid = ci * 16 + si
    b = wid // NSLOT
    slot = wid % NSLOT
    r0 = slot * RPW
    nch = jnp.where(slot == NSLOT - 1, RPT // 8, RPW // 8)
    ix0 = pl.multiple_of((r0 // (N - 1)) // 8 * 8, 8)

    pltpu.sync_copy(ryp_ref.at[b, pl.ds(0, 152), :], ry_v)
    pltpu.sync_copy(rxp_ref.at[b, pl.ds(ix0, 32), :], rx_v)

    def chunk_fn(m, carry):
        mq = lax.rem(m, 2)
        rbase = pl.multiple_of(r0 + m * 8, 8)

        @pl.when(m >= 2)
        def _wait_reuse():
            pltpu.make_async_copy(
                stage_v.at[mq],
                out_ref.at[b, pl.ds(rbase, 8), :],
                sems.at[mq],
            ).wait()

        i0 = rbase // (N - 1)
        j0 = rbase - i0 * (N - 1)
        vals = None
        for q in range(8):
            w = (j0 + q >= N - 1).astype(jnp.int32)
            i = i0 + w
            j = j0 + q - w * (N - 1)
            sry = j + (j >= i).astype(jnp.int32)
            srx = i - ix0
            nxt = []
            for idx, (off, kind) in enumerate(CHUNKS):
                if vals is not None:
                    stage_v[mq, q - 1, pl.ds(off, 16)] = vals[idx]
                if kind == 'x':
                    v = rx_v[srx, pl.ds(off, 16)]
                elif kind == 'y':
                    v = ry_v[sry, pl.ds(off, 16)]
                else:
                    v = (rx_v[srx, pl.ds(off, 16)]
                         + ry_v[sry, pl.ds(off, 16)])
                nxt.append(v)
            vals = nxt
        for idx, (off, _) in enumerate(CHUNKS):
            stage_v[mq, 7, pl.ds(off, 16)] = vals[idx]
        pltpu.make_async_copy(
            stage_v.at[mq],
            out_ref.at[b, pl.ds(rbase, 8), :],
            sems.at[mq],
        ).start()
        return carry

    lax.fori_loop(0, nch, chunk_fn, 0)
    for mq in range(2):
        pltpu.make_async_copy(
            stage_v.at[mq],
            out_ref.at[b, pl.ds(r0, 8), :],
            sems.at[mq],
        ).wait()


def _tail_body(logits_ref, boxes_ref, _big_ref, out_ref):
    rx, ry = _feature_rows(logits_ref[0], boxes_ref[0])
    lo = ry[0:N - 1, :]
    hi = ry[1:N, :]
    k = lax.broadcasted_iota(jnp.int32, (N - 1, W), 0)
    for di in range(IPB):
        i = min(TIDX * IPB + di, N - 1)
        rowx = rx[i, :].reshape(1, W)
        sub = jnp.where(k < i, lo, hi) + rowx
        out_ref[0, pl.ds(di * (N - 1), N - 1), :] = sub


def kernel(class_logits, pred_bboxes):
    rxp, ryp = pl.pallas_call(
        _table_body,
        grid=(B,),
        in_specs=[
            pl.BlockSpec((1, N, C), lambda b: (b, 0, 0)),
            pl.BlockSpec((1, N, 4), lambda b: (b, 0, 0)),
        ],
        out_specs=[
            pl.BlockSpec((1, NP, WV), lambda b: (b, 0, 0)),
            pl.BlockSpec((1, NP, WV), lambda b: (b, 0, 0)),
        ],
        out_shape=[
            jax.ShapeDtypeStruct((B, NP, WV), jnp.float32),
            jax.ShapeDtypeStruct((B, NP, WV), jnp.float32),
        ],
    )(class_logits, pred_bboxes)

    sc_expand = pl.kernel(
        _sc_expand,
        out_type=jax.ShapeDtypeStruct((B, P, W), jnp.float32),
        mesh=plsc.VectorSubcoreMesh(core_axis_name="c", subcore_axis_name="s"),
        scratch_types=[
            pltpu.VMEM((152, WV), jnp.float32),
            pltpu.VMEM((32, WV), jnp.float32),
            pltpu.VMEM((2, 8, W), jnp.float32),
            pltpu.SemaphoreType.DMA((2,)),
        ],
    )
    big = sc_expand(rxp, ryp)

    out = pl.pallas_call(
        _tail_body,
        grid=(B,),
        in_specs=[
            pl.BlockSpec((1, N, C), lambda b: (b, 0, 0)),
            pl.BlockSpec((1, N, 4), lambda b: (b, 0, 0)),
            pl.BlockSpec(memory_space=pl.ANY),
        ],
        out_specs=pl.BlockSpec((1, RPB, W), lambda b: (b, TIDX, 0)),
        out_shape=jax.ShapeDtypeStruct((B, P, W), jnp.float32),
        input_output_aliases={2: 0},
    )(class_logits, pred_bboxes, big)
    return out


# speedup vs baseline: 1.8288x; 1.0046x over previous
import jax
import jax.numpy as jnp
from jax import lax
from jax.experimental import pallas as pl
from jax.experimental.pallas import tpu as pltpu
from jax.experimental.pallas import tpu_sc as plsc

B = 4
N = 150
C = 150
P = N * (N - 1)
W = 2 * (4 + C + C + 3)

NP = 160
WV = 624
NSLOT = 8
CUT = 21456
RPW = 2688
RPT = CUT - 7 * RPW
IPB = 8
RPB = IPB * (N - 1)
TIDX = CUT // RPB

_XR = ((0, 4), (8, 158), (308, 458), (608, 609), (610, 611), (612, 613))
_YR = ((4, 8), (158, 308), (458, 608), (609, 610), (611, 612), (613, 614))


def _kind(off):
    if any(a <= off and off + 16 <= b for a, b in _XR):
        return 'x'
    if any(a <= off and off + 16 <= b for a, b in _YR):
        return 'y'
    return 'm'


CHUNKS = tuple((off, _kind(off))
               for off in tuple(range(0, 608, 16)) + (598,))
_XOFFS = tuple(off for off, kind in CHUNKS if kind != 'y')
XPOS = {off: 16 * n for n, off in enumerate(_XOFFS)}
WX = 16 * len(_XOFFS)


def _feature_rows(logits, box):
    s = jax.nn.sigmoid(logits)
    soft = jnp.minimum(1.0 - s, s)
    m = jnp.max(s, axis=-1, keepdims=True)
    lp = jnp.log(m + 1e-08)
    ln = jnp.log(1.0 - m + 1e-08)
    z4 = jnp.zeros((N, 4), jnp.float32)
    zC = jnp.zeros((N, C), jnp.float32)
    z1 = jnp.zeros((N, 1), jnp.float32)
    rx = jnp.concatenate(
        [box, z4, s, zC, soft, zC, lp, z1, ln, z1, m, z1], axis=-1)
    ry = jnp.concatenate(
        [z4, box, zC, s, zC, soft, z1, lp, z1, ln, z1, m], axis=-1)
    return rx, ry


def _table_body(logits_ref, boxes_ref, rx_ref, ry_ref):
    rx, ry = _feature_rows(logits_ref[0], boxes_ref[0])
    zpad = jnp.zeros((N, WV - W), jnp.float32)
    rxpad = jnp.pad(rx, ((0, 0), (0, 16 - (W - 608))))
    rxc = jnp.concatenate([rxpad[:, off:off + 16] for off in _XOFFS],
                          axis=-1)
    rx_ref[0, 0:N, :] = rxc
    ry_ref[0, 0:N, :] = jnp.concatenate([ry, zpad], axis=-1)
    rx_ref[0, N:NP, :] = jnp.zeros((NP - N, WX), jnp.float32)
    ry_ref[0, N:NP, :] = jnp.zeros((NP - N, WV), jnp.float32)


def _sc_expand(rxp_ref, ryp_ref, out_ref, ry_v, rx_v, stage_v, sems):
    ci = lax.axis_index("c")
    si = lax.axis_index("s")
    wid = ci * 16 + si
    b = wid // NSLOT
    slot = wid % NSLOT
    r0 = slot * RPW
    nch = jnp.where(slot == NSLOT - 1, RPT // 8, RPW // 8)
    ix0 = pl.multiple_of((r0 // (N - 1)) // 8 * 8, 8)

    pltpu.sync_copy(ryp_ref.at[b, pl.ds(0, 152), :], ry_v)
    pltpu.sync_copy(rxp_ref.at[b, pl.ds(ix0, 32), :], rx_v)

    def chunk_fn(m, carry):
        mq = lax.rem(m, 3)
        rbase = pl.multiple_of(r0 + m * 8, 8)

        @pl.when(m >= 3)
        def _wait_reuse():
            pltpu.make_async_copy(
                stage_v.at[mq],
                out_ref.at[b, pl.ds(rbase, 8), :],
                sems.at[mq],
            ).wait()

        i0 = rbase // (N - 1)
        j0 = rbase - i0 * (N - 1)
        vals = None
        for q in range(8):
            w = (j0 + q >= N - 1).astype(jnp.int32)
            i = i0 + w
            j = j0 + q - w * (N - 1)
            sry = j + (j >= i).astype(jnp.int32)
            srx = i - ix0
            nxt = []
            for idx, (off, kind) in enumerate(CHUNKS):
                if vals is not None:
                    stage_v[mq, q - 1, pl.ds(off, 16)] = vals[idx]
                if kind == 'x':
                    v = rx_v[srx, pl.ds(XPOS[off], 16)]
                elif kind == 'y':
                    v = ry_v[sry, pl.ds(off, 16)]
                else:
                    v = (rx_v[srx, pl.ds(XPOS[off], 16)]
                         + ry_v[sry, pl.ds(off, 16)])
                nxt.append(v)
            vals = nxt
        for idx, (off, _) in enumerate(CHUNKS):
            stage_v[mq, 7, pl.ds(off, 16)] = vals[idx]
        pltpu.make_async_copy(
            stage_v.at[mq],
            out_ref.at[b, pl.ds(rbase, 8), :],
            sems.at[mq],
        ).start()
        return carry

    lax.fori_loop(0, nch, chunk_fn, 0)
    for mq in range(3):
        pltpu.make_async_copy(
            stage_v.at[mq],
            out_ref.at[b, pl.ds(r0, 8), :],
            sems.at[mq],
        ).wait()


def _tail_body(logits_ref, boxes_ref, _big_ref, out_ref):
    rx, ry = _feature_rows(logits_ref[0], boxes_ref[0])
    lo = ry[0:N - 1, :]
    hi = ry[1:N, :]
    k = lax.broadcasted_iota(jnp.int32, (N - 1, W), 0)
    for di in range(IPB):
        i = min(TIDX * IPB + di, N - 1)
        rowx = rx[i, :].reshape(1, W)
        sub = jnp.where(k < i, lo, hi) + rowx
        out_ref[0, pl.ds(di * (N - 1), N - 1), :] = sub


def kernel(class_logits, pred_bboxes):
    rxp, ryp = pl.pallas_call(
        _table_body,
        grid=(B,),
        in_specs=[
            pl.BlockSpec((1, N, C), lambda b: (b, 0, 0)),
            pl.BlockSpec((1, N, 4), lambda b: (b, 0, 0)),
        ],
        out_specs=[
            pl.BlockSpec((1, NP, WX), lambda b: (b, 0, 0)),
            pl.BlockSpec((1, NP, WV), lambda b: (b, 0, 0)),
        ],
        out_shape=[
            jax.ShapeDtypeStruct((B, NP, WX), jnp.float32),
            jax.ShapeDtypeStruct((B, NP, WV), jnp.float32),
        ],
    )(class_logits, pred_bboxes)

    sc_expand = pl.kernel(
        _sc_expand,
        out_type=jax.ShapeDtypeStruct((B, P, W), jnp.float32),
        mesh=plsc.VectorSubcoreMesh(core_axis_name="c", subcore_axis_name="s"),
        scratch_types=[
            pltpu.VMEM((152, WV), jnp.float32),
            pltpu.VMEM((32, WX), jnp.float32),
            pltpu.VMEM((3, 8, W), jnp.float32),
            pltpu.SemaphoreType.DMA((2,)),
        ],
    )
    big = sc_expand(rxp, ryp)

    out = pl.pallas_call(
        _tail_body,
        grid=(B,),
        in_specs=[
            pl.BlockSpec((1, N, C), lambda b: (b, 0, 0)),
            pl.BlockSpec((1, N, 4), lambda b: (b, 0, 0)),
            pl.BlockSpec(memory_space=pl.ANY),
        ],
        out_specs=pl.BlockSpec((1, RPB, W), lambda b: (b, TIDX, 0)),
        out_shape=jax.ShapeDtypeStruct((B, P, W), jnp.float32),
        input_output_aliases={2: 0},
    )(class_logits, pred_bboxes, big)
    return out
